# baseline (speedup 1.0000x reference)
import functools

import jax
import jax.numpy as jnp
from jax import lax
from jax.experimental import pallas as pl
from jax.experimental.pallas import tpu as pltpu
from jax.experimental.pallas import tpu_sc as plsc

B = 16384
N_ANIMAL, D_ANIMAL = 80, 5
N_ITEM, D_ITEM = 20, 3
NC, NS, L = 2, 16, 16
NW = NC * NS
BPW = B // NW
CHUNKS = BPW // L


@functools.cache
def _build_sc_unit():
  mesh = plsc.VectorSubcoreMesh(
      core_axis_name="c", subcore_axis_name="s", num_cores=NC, num_subcores=NS
  )

  @functools.partial(
      pl.kernel,
      out_type=(
          jax.ShapeDtypeStruct((B * D_ANIMAL,), jnp.float32),
          jax.ShapeDtypeStruct((B * D_ITEM,), jnp.float32),
          jax.ShapeDtypeStruct((B * 2,), jnp.float32),
      ),
      mesh=mesh,
      scratch_types=[
          pltpu.VMEM((BPW,), jnp.int32),
          pltpu.VMEM((BPW,), jnp.int32),
          pltpu.VMEM((2 * BPW,), jnp.float32),
          pltpu.VMEM((N_ANIMAL * D_ANIMAL,), jnp.float32),
          pltpu.VMEM((N_ITEM * D_ITEM,), jnp.float32),
          pltpu.VMEM((6 * L,), jnp.float32),
          pltpu.VMEM((D_ANIMAL * BPW,), jnp.float32),
          pltpu.VMEM((D_ITEM * BPW,), jnp.float32),
          pltpu.VMEM((2 * BPW,), jnp.float32),
          pltpu.SemaphoreType.DMA,
      ],
      compiler_params=pltpu.CompilerParams(needs_layout_passes=False),
  )
  def sc_unit(aid_hbm, iid_hbm, hp_hbm, ta_hbm, ti_hbm, wb_hbm,
              oa_hbm, oi_hbm, os_hbm,
              aid_v, iid_v, hp_v, ta_v, ti_v, wb_v, oa_v, oi_v, os_v,
              sem):
    wid = lax.axis_index("s") * NC + lax.axis_index("c")
    base = wid * BPW
    cps = [
        pltpu.async_copy(aid_hbm.at[pl.ds(base, BPW)], aid_v, sem),
        pltpu.async_copy(iid_hbm.at[pl.ds(base, BPW)], iid_v, sem),
        pltpu.async_copy(hp_hbm.at[pl.ds(base * 2, 2 * BPW)], hp_v, sem),
        pltpu.async_copy(ta_hbm, ta_v, sem),
        pltpu.async_copy(ti_hbm, ti_v, sem),
        pltpu.async_copy(wb_hbm, wb_v, sem),
    ]
    for cp in cps:
      cp.wait()

    w00 = wb_v[pl.ds(0 * L, L)]
    w01 = wb_v[pl.ds(1 * L, L)]
    w10 = wb_v[pl.ds(2 * L, L)]
    w11 = wb_v[pl.ds(3 * L, L)]
    b0 = wb_v[pl.ds(4 * L, L)]
    b1 = wb_v[pl.ds(5 * L, L)]

    @plsc.parallel_loop(0, CHUNKS, unroll=4)
    def body(c):
      rows = c * L + lax.iota(jnp.int32, L)
      aid = aid_v[pl.ds(c * L, L)]
      iid = iid_v[pl.ds(c * L, L)]
      for d in range(D_ANIMAL):
        v = plsc.load_gather(ta_v, [aid * D_ANIMAL + d])
        plsc.store_scatter(oa_v, [rows * D_ANIMAL + d], v)
      for d in range(D_ITEM):
        v = plsc.load_gather(ti_v, [iid * D_ITEM + d])
        plsc.store_scatter(oi_v, [rows * D_ITEM + d], v)
      hp = plsc.load_gather(hp_v, [rows * 2])
      atk = plsc.load_gather(hp_v, [rows * 2 + 1])
      plsc.store_scatter(os_v, [rows * 2], hp * w00 + atk * w01 + b0)
      plsc.store_scatter(os_v, [rows * 2 + 1], hp * w10 + atk * w11 + b1)

    ocps = [
        pltpu.async_copy(oa_v, oa_hbm.at[pl.ds(base * D_ANIMAL, D_ANIMAL * BPW)], sem),
        pltpu.async_copy(oi_v, oi_hbm.at[pl.ds(base * D_ITEM, D_ITEM * BPW)], sem),
        pltpu.async_copy(os_v, os_hbm.at[pl.ds(base * 2, 2 * BPW)], sem),
    ]
    for cp in ocps:
      cp.wait()

  return sc_unit


def kernel(animal_id, item_id, hp_atk, table_animal, table_item, W, b):
  oa, oi, os_ = _build_sc_unit()(
      animal_id,
      item_id,
      hp_atk.reshape(-1),
      table_animal.reshape(-1),
      table_item.reshape(-1),
      jnp.broadcast_to(
          jnp.concatenate([W.reshape(-1), b]).reshape(6, 1), (6, L)
      ).reshape(-1),
  )
  return (oa.reshape(B, D_ANIMAL), oi.reshape(B, D_ITEM), os_.reshape(B, 2))


# speedup vs baseline: 3.0905x; 3.0905x over previous
import functools

import jax
import jax.numpy as jnp
from jax import lax
from jax.experimental import pallas as pl
from jax.experimental.pallas import tpu as pltpu
from jax.experimental.pallas import tpu_sc as plsc

B = 16384
N_ANIMAL, D_ANIMAL = 80, 5
N_ITEM, D_ITEM = 20, 3
NC, NS, L = 2, 16, 16
NW = NC * NS
BPW = B // NW
CHUNKS = BPW // L


@functools.cache
def _build_sc_unit():
  mesh = plsc.VectorSubcoreMesh(
      core_axis_name="c", subcore_axis_name="s", num_cores=NC, num_subcores=NS
  )

  @functools.partial(
      pl.kernel,
      out_type=(
          jax.ShapeDtypeStruct((D_ANIMAL, B), jnp.float32),
          jax.ShapeDtypeStruct((D_ITEM, B), jnp.float32),
          jax.ShapeDtypeStruct((2, B), jnp.float32),
      ),
      mesh=mesh,
      scratch_types=[
          pltpu.VMEM((BPW,), jnp.int32),
          pltpu.VMEM((BPW,), jnp.int32),
          pltpu.VMEM((BPW,), jnp.float32),
          pltpu.VMEM((BPW,), jnp.float32),
          pltpu.VMEM((N_ANIMAL * D_ANIMAL,), jnp.float32),
          pltpu.VMEM((N_ITEM * D_ITEM,), jnp.float32),
          pltpu.VMEM((6 * L,), jnp.float32),
          pltpu.VMEM((D_ANIMAL * BPW,), jnp.float32),
          pltpu.VMEM((D_ITEM * BPW,), jnp.float32),
          pltpu.VMEM((2 * BPW,), jnp.float32),
          pltpu.SemaphoreType.DMA,
      ],
      compiler_params=pltpu.CompilerParams(
          needs_layout_passes=False, use_tc_tiling_on_sc=False
      ),
  )
  def sc_unit(aid_hbm, iid_hbm, hp_hbm, ta_hbm, ti_hbm, wb_hbm,
              oa_hbm, oi_hbm, os_hbm,
              aid_v, iid_v, hp_v, atk_v, ta_v, ti_v, wb_v, oa_v, oi_v, os_v,
              sem):
    wid = lax.axis_index("s") * NC + lax.axis_index("c")
    base = wid * BPW
    cps = [
        pltpu.async_copy(aid_hbm.at[pl.ds(base, BPW)], aid_v, sem),
        pltpu.async_copy(iid_hbm.at[pl.ds(base, BPW)], iid_v, sem),
        pltpu.async_copy(hp_hbm.at[pl.ds(base, BPW)], hp_v, sem),
        pltpu.async_copy(hp_hbm.at[pl.ds(B + base, BPW)], atk_v, sem),
        pltpu.async_copy(ta_hbm, ta_v, sem),
        pltpu.async_copy(ti_hbm, ti_v, sem),
        pltpu.async_copy(wb_hbm, wb_v, sem),
    ]
    for cp in cps:
      cp.wait()

    w00 = wb_v[pl.ds(0 * L, L)]
    w01 = wb_v[pl.ds(1 * L, L)]
    w10 = wb_v[pl.ds(2 * L, L)]
    w11 = wb_v[pl.ds(3 * L, L)]
    b0 = wb_v[pl.ds(4 * L, L)]
    b1 = wb_v[pl.ds(5 * L, L)]

    @plsc.parallel_loop(0, CHUNKS, unroll=4)
    def body(c):
      off = c * L
      aid = aid_v[pl.ds(off, L)]
      iid = iid_v[pl.ds(off, L)]
      for d in range(D_ANIMAL):
        oa_v[pl.ds(d * BPW + off, L)] = plsc.load_gather(
            ta_v, [aid * D_ANIMAL + d])
      for d in range(D_ITEM):
        oi_v[pl.ds(d * BPW + off, L)] = plsc.load_gather(
            ti_v, [iid * D_ITEM + d])
      hp = hp_v[pl.ds(off, L)]
      atk = atk_v[pl.ds(off, L)]
      os_v[pl.ds(off, L)] = hp * w00 + atk * w01 + b0
      os_v[pl.ds(BPW + off, L)] = hp * w10 + atk * w11 + b1

    ocps = []
    for d in range(D_ANIMAL):
      ocps.append(pltpu.async_copy(
          oa_v.at[pl.ds(d * BPW, BPW)], oa_hbm.at[d, pl.ds(base, BPW)], sem))
    for d in range(D_ITEM):
      ocps.append(pltpu.async_copy(
          oi_v.at[pl.ds(d * BPW, BPW)], oi_hbm.at[d, pl.ds(base, BPW)], sem))
    for d in range(2):
      ocps.append(pltpu.async_copy(
          os_v.at[pl.ds(d * BPW, BPW)], os_hbm.at[d, pl.ds(base, BPW)], sem))
    for cp in ocps:
      cp.wait()

  return sc_unit


def kernel(animal_id, item_id, hp_atk, table_animal, table_item, W, b):
  wb = jnp.broadcast_to(
      jnp.concatenate([W.reshape(-1), b]).reshape(6, 1), (6, L)
  ).reshape(-1)
  oa_t, oi_t, os_t = _build_sc_unit()(
      animal_id,
      item_id,
      hp_atk.T.reshape(-1),
      table_animal.reshape(-1),
      table_item.reshape(-1),
      wb,
  )
  return (oa_t.T, oi_t.T, os_t.T)
